# baseline (speedup 1.0000x reference)
import functools

import jax
import jax.numpy as jnp
from jax import lax
from jax.experimental import pallas as pl
from jax.experimental.pallas import tpu as pltpu
from jax.experimental.pallas import tpu_sc as plsc

N = 10000
E = 320000
D = 128
NC, NS = 2, 16
NW = NC * NS
CG = 64
CS = 128
EPW = 10240
E_PAD = NW * EPW
NPH = 2
GPP = EPW // NPH // CG
SPP = EPW // NPH // CS
NSLOT = 4
NP = 10240
RPT = NP // NS

_mesh = plsc.VectorSubcoreMesh(core_axis_name="c", subcore_axis_name="s")


@functools.partial(
    pl.kernel,
    out_type=jax.ShapeDtypeStruct((NC, NP), jnp.float32),
    mesh=_mesh,
    scratch_types=[
        pltpu.VMEM((EPW // CS, CS), jnp.int32),
        pltpu.VMEM((CS,), jnp.float32),
        pltpu.VMEM((RPT,), jnp.float32),
        pltpu.VMEM_SHARED((NP,), jnp.float32),
        pltpu.SemaphoreType.DMA,
    ],
)
def _deg(src_hbm, out_hbm, idx_v, ones_v, zero_v, acc_sh, dsem):
    cid = lax.axis_index("c")
    sid = lax.axis_index("s")
    wid = cid * NS + sid
    cw = EPW // CS

    one16 = jnp.ones((16,), jnp.float32)
    z16 = jnp.zeros((16,), jnp.float32)
    for k in range(CS // 16):
        ones_v[pl.ds(k * 16, 16)] = one16

    def zbody(k, _):
        zero_v[pl.ds(k * 16, 16)] = z16
        return 0

    lax.fori_loop(0, RPT // 16, zbody, 0)
    pltpu.sync_copy(zero_v, acc_sh.at[pl.ds(sid * RPT, RPT)])
    pltpu.sync_copy(src_hbm.at[pl.ds(wid * cw, cw)], idx_v)
    plsc.subcore_barrier()

    def body(j, _):
        pltpu.sync_copy(ones_v, acc_sh.at[idx_v.at[j]], add=True)
        return 0

    lax.fori_loop(0, cw, body, 0)
    plsc.subcore_barrier()
    pltpu.sync_copy(acc_sh.at[pl.ds(sid * RPT, RPT)],
                    out_hbm.at[cid, pl.ds(sid * RPT, RPT)])


@functools.partial(
    pl.kernel,
    out_type=jax.ShapeDtypeStruct((NC, NP, D), jnp.float32),
    mesh=_mesh,
    scratch_types=[
        pltpu.VMEM((GPP, CG), jnp.int32),
        pltpu.VMEM((SPP, CS), jnp.int32),
        pltpu.VMEM((NSLOT * CG, D), jnp.float32),
        pltpu.VMEM_SHARED((NP, D), jnp.float32),
        [pltpu.SemaphoreType.DMA] * NSLOT,
        [pltpu.SemaphoreType.DMA] * (NSLOT // 2),
    ],
)
def _hop(g_hbm, src_hbm, dst_hbm, out_hbm,
         src_v, dst_v, ring_v, acc_sh, gsems, ssems):
    cid = lax.axis_index("c")
    sid = lax.axis_index("s")
    wid = cid * NS + sid

    z16 = jnp.zeros((16,), jnp.float32)

    def zrow(r, _):
        for k in range(D // 16):
            ring_v[r, pl.ds(k * 16, 16)] = z16
        return 0

    lax.fori_loop(0, CS, zrow, 0)

    def zcp(cnk, _):
        pltpu.async_copy(ring_v.at[pl.ds(0, CS)],
                         acc_sh.at[pl.ds(sid * RPT + cnk * CS, CS)],
                         gsems[0])
        return 0

    lax.fori_loop(0, RPT // CS, zcp, 0)

    def zwait(cnk, _):
        pltpu.make_async_copy(ring_v.at[pl.ds(0, CS)],
                              acc_sh.at[pl.ds(sid * RPT, CS)],
                              gsems[0]).wait()
        return 0

    lax.fori_loop(0, RPT // CS, zwait, 0)
    plsc.subcore_barrier()

    def slot(s):
        return ring_v.at[pl.ds(s * CG, CG)]

    def gather(c, s):
        pltpu.async_copy(g_hbm.at[src_v.at[c]], slot(s), gsems[s])

    for ph in range(NPH):
        pltpu.sync_copy(src_hbm.at[pl.ds((wid * NPH + ph) * GPP, GPP)],
                        src_v)
        pltpu.sync_copy(dst_hbm.at[pl.ds((wid * NPH + ph) * SPP, SPP)],
                        dst_v)
        for s in range(NSLOT):
            gather(s, s)

        def ring(rr, _):
            c0 = NSLOT * rr
            for pr in range(NSLOT // 2):
                s0 = 2 * pr
                c = c0 + 2 * pr
                pltpu.make_async_copy(
                    g_hbm.at[src_v.at[c]], slot(s0), gsems[s0]).wait()
                pltpu.make_async_copy(
                    g_hbm.at[src_v.at[c]], slot(s0 + 1),
                    gsems[s0 + 1]).wait()
                pltpu.async_copy(ring_v.at[pl.ds(s0 * CG, CS)],
                                 acc_sh.at[dst_v.at[c // 2]],
                                 ssems[pr], add=True)
            for pr in range(NSLOT // 2):
                s0 = 2 * pr
                c = c0 + 2 * pr
                pltpu.make_async_copy(ring_v.at[pl.ds(s0 * CG, CS)],
                                      acc_sh.at[dst_v.at[c // 2]],
                                      ssems[pr]).wait()
                gather(jnp.minimum(c + NSLOT, GPP - 2), s0)
                gather(jnp.minimum(c + 1 + NSLOT, GPP - 1), s0 + 1)
            return 0

        lax.fori_loop(0, GPP // NSLOT, ring, 0)
        for s in range(NSLOT):
            pltpu.make_async_copy(
                g_hbm.at[src_v.at[0]], slot(s), gsems[s]).wait()

    plsc.subcore_barrier()

    def drain(cnk, _):
        base = sid * RPT + cnk * CS
        pltpu.sync_copy(acc_sh.at[pl.ds(base, CS)],
                        out_hbm.at[cid, pl.ds(base, CS)])
        return 0

    lax.fori_loop(0, RPT // CS, drain, 0)


_RB = 512


def _mm_body(x_ref, w_ref, b_ref, degp_ref, h_ref, g_ref):
    i = pl.program_id(0)
    h = jnp.dot(x_ref[...], w_ref[...],
                preferred_element_type=jnp.float32) + b_ref[...]
    db = (degp_ref[0, pl.ds(i * _RB, _RB)]
          + degp_ref[1, pl.ds(i * _RB, _RB)] + 1.0)
    norm = lax.rsqrt(db).reshape(_RB, 1)
    h_ref[...] = h
    g_ref[...] = h * norm


def _comb_body(accp_ref, h_ref, degp_ref, hn_ref, gn_ref):
    i = pl.program_id(0)
    acc = accp_ref[0] + accp_ref[1]
    db = (degp_ref[0, pl.ds(i * _RB, _RB)]
          + degp_ref[1, pl.ds(i * _RB, _RB)] + 1.0).reshape(_RB, 1)
    norm = lax.rsqrt(db)
    h = h_ref[...]
    hn = acc * norm + h / db
    hn_ref[...] = hn
    gn_ref[...] = hn * norm


def _comb_last_body(accp_ref, h_ref, degp_ref, hn_ref):
    i = pl.program_id(0)
    acc = accp_ref[0] + accp_ref[1]
    db = (degp_ref[0, pl.ds(i * _RB, _RB)]
          + degp_ref[1, pl.ds(i * _RB, _RB)] + 1.0).reshape(_RB, 1)
    hn_ref[...] = acc * lax.rsqrt(db) + h_ref[...] / db


_row_spec = pl.BlockSpec((_RB, D), lambda i: (i, 0))
_degp_spec = pl.BlockSpec((NC, NP), lambda i: (0, 0))
_out2 = [jax.ShapeDtypeStruct((NP, D), jnp.float32)] * 2

_mm_call = pl.pallas_call(
    _mm_body,
    grid=(NP // _RB,),
    in_specs=[
        _row_spec,
        pl.BlockSpec((D, D), lambda i: (0, 0)),
        pl.BlockSpec((1, D), lambda i: (0, 0)),
        _degp_spec,
    ],
    out_specs=[_row_spec, _row_spec],
    out_shape=_out2,
)

_comb_call = pl.pallas_call(
    _comb_body,
    grid=(NP // _RB,),
    in_specs=[
        pl.BlockSpec((NC, _RB, D), lambda i: (0, i, 0)),
        _row_spec,
        _degp_spec,
    ],
    out_specs=[_row_spec, _row_spec],
    out_shape=_out2,
)

_comb_last_call = pl.pallas_call(
    _comb_last_body,
    grid=(NP // _RB,),
    in_specs=[
        pl.BlockSpec((NC, _RB, D), lambda i: (0, i, 0)),
        _row_spec,
        _degp_spec,
    ],
    out_specs=[_row_spec],
    out_shape=[jax.ShapeDtypeStruct((NP, D), jnp.float32)],
)


def kernel(x, edge_index, W, b):
    src = edge_index[0]
    dst = edge_index[1]
    pad = N + jnp.arange(E_PAD - E, dtype=jnp.int32) % (NP - N)
    src_pad = jnp.concatenate([src, pad])
    srcg = src_pad.reshape(E_PAD // CG, CG)
    srcs = src_pad.reshape(E_PAD // CS, CS)
    dstr = jnp.concatenate([dst, pad]).reshape(E_PAD // CS, CS)
    x_pad = jnp.pad(x, ((0, NP - N), (0, 0)))

    degp = _deg(srcs)
    h, g = _mm_call(x_pad, W, b.reshape(1, D), degp)
    accp = _hop(g, srcg, dstr)
    h, g = _comb_call(accp, h, degp)
    accp = _hop(g, srcg, dstr)
    (h,) = _comb_last_call(accp, h, degp)
    return h[:N]


# speedup vs baseline: 1.2004x; 1.2004x over previous
import functools

import jax
import jax.numpy as jnp
from jax import lax
from jax.experimental import pallas as pl
from jax.experimental.pallas import tpu as pltpu
from jax.experimental.pallas import tpu_sc as plsc

N = 10000
E = 320000
D = 128
NC, NS = 2, 16
NW = NC * NS
CG = 64
CS = 128
EPW = 10240
E_PAD = NW * EPW
NPH = 2
GPP = EPW // NPH // CG
SPP = EPW // NPH // CS
NSLOT = 4
NP = 10240
RPT = NP // NS

_mesh = plsc.VectorSubcoreMesh(core_axis_name="c", subcore_axis_name="s")


@functools.partial(
    pl.kernel,
    out_type=jax.ShapeDtypeStruct((NC, NP), jnp.float32),
    mesh=_mesh,
    scratch_types=[
        pltpu.VMEM((EPW // CS, CS), jnp.int32),
        pltpu.VMEM((CS,), jnp.float32),
        pltpu.VMEM((RPT,), jnp.float32),
        pltpu.VMEM_SHARED((NP,), jnp.float32),
        pltpu.SemaphoreType.DMA,
    ],
)
def _deg(src_hbm, out_hbm, idx_v, ones_v, zero_v, acc_sh, dsem):
    cid = lax.axis_index("c")
    sid = lax.axis_index("s")
    wid = cid * NS + sid
    cw = EPW // CS

    one16 = jnp.ones((16,), jnp.float32)
    z16 = jnp.zeros((16,), jnp.float32)
    for k in range(CS // 16):
        ones_v[pl.ds(k * 16, 16)] = one16

    def zbody(k, _):
        zero_v[pl.ds(k * 16, 16)] = z16
        return 0

    lax.fori_loop(0, RPT // 16, zbody, 0)
    pltpu.sync_copy(zero_v, acc_sh.at[pl.ds(sid * RPT, RPT)])
    pltpu.sync_copy(src_hbm.at[pl.ds(wid * cw, cw)], idx_v)
    plsc.subcore_barrier()

    def body(j, _):
        pltpu.sync_copy(ones_v, acc_sh.at[idx_v.at[j]], add=True)
        return 0

    lax.fori_loop(0, cw, body, 0)
    plsc.subcore_barrier()
    pltpu.sync_copy(acc_sh.at[pl.ds(sid * RPT, RPT)],
                    out_hbm.at[cid, pl.ds(sid * RPT, RPT)])


@functools.partial(
    pl.kernel,
    out_type=jax.ShapeDtypeStruct((NC, NP, D), jnp.float32),
    mesh=_mesh,
    scratch_types=[
        pltpu.VMEM((GPP, CG), jnp.int32),
        pltpu.VMEM((SPP, CS), jnp.int32),
        pltpu.VMEM((NSLOT * CG, D), jnp.float32),
        pltpu.VMEM_SHARED((NP, D), jnp.float32),
        [pltpu.SemaphoreType.DMA] * NSLOT,
        [pltpu.SemaphoreType.DMA] * (NSLOT // 2),
    ],
)
def _hop(g_hbm, src_hbm, dst_hbm, out_hbm,
         src_v, dst_v, ring_v, acc_sh, gsems, ssems):
    cid = lax.axis_index("c")
    sid = lax.axis_index("s")
    wid = cid * NS + sid

    z16 = jnp.zeros((16,), jnp.float32)

    def zrow(r, _):
        for k in range(D // 16):
            ring_v[r, pl.ds(k * 16, 16)] = z16
        return 0

    lax.fori_loop(0, CS, zrow, 0)

    def zcp(cnk, _):
        pltpu.async_copy(ring_v.at[pl.ds(0, CS)],
                         acc_sh.at[pl.ds(sid * RPT + cnk * CS, CS)],
                         gsems[0])
        return 0

    lax.fori_loop(0, RPT // CS, zcp, 0)

    def zwait(cnk, _):
        pltpu.make_async_copy(ring_v.at[pl.ds(0, CS)],
                              acc_sh.at[pl.ds(sid * RPT, CS)],
                              gsems[0]).wait()
        return 0

    lax.fori_loop(0, RPT // CS, zwait, 0)
    plsc.subcore_barrier()

    def slot(s):
        return ring_v.at[pl.ds(s * CG, CG)]

    def gather(c, s):
        pltpu.async_copy(g_hbm.at[src_v.at[c]], slot(s), gsems[s])

    for ph in range(NPH):
        pltpu.sync_copy(src_hbm.at[pl.ds((wid * NPH + ph) * GPP, GPP)],
                        src_v)
        pltpu.sync_copy(dst_hbm.at[pl.ds((wid * NPH + ph) * SPP, SPP)],
                        dst_v)
        for s in range(NSLOT):
            gather(s, s)

        def ring(rr, _):
            for pr in range(NSLOT // 2):
                s0, s1 = 2 * pr, 2 * pr + 1
                c0 = NSLOT * rr + 2 * pr
                k = c0 // 2
                pltpu.make_async_copy(
                    g_hbm.at[src_v.at[c0]], slot(s0), gsems[s0]).wait()
                pltpu.make_async_copy(
                    g_hbm.at[src_v.at[c0]], slot(s1), gsems[s1]).wait()
                pair = ring_v.at[pl.ds(s0 * CG, CS)]
                pltpu.async_copy(pair, acc_sh.at[dst_v.at[k]],
                                 ssems[pr], add=True)
                pltpu.make_async_copy(pair, acc_sh.at[dst_v.at[k]],
                                      ssems[pr]).wait()
                gather(jnp.minimum(c0 + NSLOT, GPP - 2), s0)
                gather(jnp.minimum(c0 + 1 + NSLOT, GPP - 1), s1)
            return 0

        lax.fori_loop(0, GPP // NSLOT, ring, 0)
        for s in range(NSLOT):
            pltpu.make_async_copy(
                g_hbm.at[src_v.at[0]], slot(s), gsems[s]).wait()

    plsc.subcore_barrier()

    def drain(cnk, _):
        base = sid * RPT + cnk * CS
        pltpu.sync_copy(acc_sh.at[pl.ds(base, CS)],
                        out_hbm.at[cid, pl.ds(base, CS)])
        return 0

    lax.fori_loop(0, RPT // CS, drain, 0)


_RB = 512


def _mm_body(x_ref, w_ref, b_ref, degp_ref, h_ref, g_ref):
    i = pl.program_id(0)
    h = jnp.dot(x_ref[...], w_ref[...],
                preferred_element_type=jnp.float32) + b_ref[...]
    db = (degp_ref[0, pl.ds(i * _RB, _RB)]
          + degp_ref[1, pl.ds(i * _RB, _RB)] + 1.0)
    norm = lax.rsqrt(db).reshape(_RB, 1)
    h_ref[...] = h
    g_ref[...] = h * norm


def _comb_body(accp_ref, h_ref, degp_ref, hn_ref, gn_ref):
    i = pl.program_id(0)
    acc = accp_ref[0] + accp_ref[1]
    db = (degp_ref[0, pl.ds(i * _RB, _RB)]
          + degp_ref[1, pl.ds(i * _RB, _RB)] + 1.0).reshape(_RB, 1)
    norm = lax.rsqrt(db)
    h = h_ref[...]
    hn = acc * norm + h / db
    hn_ref[...] = hn
    gn_ref[...] = hn * norm


def _comb_last_body(accp_ref, h_ref, degp_ref, hn_ref):
    i = pl.program_id(0)
    acc = accp_ref[0] + accp_ref[1]
    db = (degp_ref[0, pl.ds(i * _RB, _RB)]
          + degp_ref[1, pl.ds(i * _RB, _RB)] + 1.0).reshape(_RB, 1)
    hn_ref[...] = acc * lax.rsqrt(db) + h_ref[...] / db


_row_spec = pl.BlockSpec((_RB, D), lambda i: (i, 0))
_degp_spec = pl.BlockSpec((NC, NP), lambda i: (0, 0))
_out2 = [jax.ShapeDtypeStruct((NP, D), jnp.float32)] * 2

_mm_call = pl.pallas_call(
    _mm_body,
    grid=(NP // _RB,),
    in_specs=[
        _row_spec,
        pl.BlockSpec((D, D), lambda i: (0, 0)),
        pl.BlockSpec((1, D), lambda i: (0, 0)),
        _degp_spec,
    ],
    out_specs=[_row_spec, _row_spec],
    out_shape=_out2,
)

_comb_call = pl.pallas_call(
    _comb_body,
    grid=(NP // _RB,),
    in_specs=[
        pl.BlockSpec((NC, _RB, D), lambda i: (0, i, 0)),
        _row_spec,
        _degp_spec,
    ],
    out_specs=[_row_spec, _row_spec],
    out_shape=_out2,
)

_comb_last_call = pl.pallas_call(
    _comb_last_body,
    grid=(NP // _RB,),
    in_specs=[
        pl.BlockSpec((NC, _RB, D), lambda i: (0, i, 0)),
        _row_spec,
        _degp_spec,
    ],
    out_specs=[_row_spec],
    out_shape=[jax.ShapeDtypeStruct((NP, D), jnp.float32)],
)


def kernel(x, edge_index, W, b):
    src = edge_index[0]
    dst = edge_index[1]
    pad = N + jnp.arange(E_PAD - E, dtype=jnp.int32) % (NP - N)
    src_pad = jnp.concatenate([src, pad])
    srcg = src_pad.reshape(E_PAD // CG, CG)
    srcs = src_pad.reshape(E_PAD // CS, CS)
    dstr = jnp.concatenate([dst, pad]).reshape(E_PAD // CS, CS)
    x_pad = jnp.pad(x, ((0, NP - N), (0, 0)))

    degp = _deg(srcs)
    h, g = _mm_call(x_pad, W, b.reshape(1, D), degp)
    accp = _hop(g, srcg, dstr)
    h, g = _comb_call(accp, h, degp)
    accp = _hop(g, srcg, dstr)
    (h,) = _comb_last_call(accp, h, degp)
    return h[:N]


# speedup vs baseline: 1.2094x; 1.0074x over previous
import functools

import jax
import jax.numpy as jnp
from jax import lax
from jax.experimental import pallas as pl
from jax.experimental.pallas import tpu as pltpu
from jax.experimental.pallas import tpu_sc as plsc

N = 10000
E = 320000
D = 128
NC, NS = 2, 16
NW = NC * NS
CG = 64
CS = 128
EPW = 10240
E_PAD = NW * EPW
NPH = 2
GPP = EPW // NPH // CG
SPP = EPW // NPH // CS
NSLOT = 4
NP = 10240
RPT = NP // NS

_mesh = plsc.VectorSubcoreMesh(core_axis_name="c", subcore_axis_name="s")


@functools.partial(
    pl.kernel,
    out_type=jax.ShapeDtypeStruct((NC, NP), jnp.float32),
    mesh=_mesh,
    scratch_types=[
        pltpu.VMEM((EPW // CS, CS), jnp.int32),
        pltpu.VMEM((CS,), jnp.float32),
        pltpu.VMEM((RPT,), jnp.float32),
        pltpu.VMEM_SHARED((NP,), jnp.float32),
        pltpu.SemaphoreType.DMA,
    ],
)
def _deg(src_hbm, out_hbm, idx_v, ones_v, zero_v, acc_sh, dsem):
    cid = lax.axis_index("c")
    sid = lax.axis_index("s")
    wid = cid * NS + sid
    cw = EPW // CS

    one16 = jnp.ones((16,), jnp.float32)
    z16 = jnp.zeros((16,), jnp.float32)
    for k in range(CS // 16):
        ones_v[pl.ds(k * 16, 16)] = one16

    def zbody(k, _):
        zero_v[pl.ds(k * 16, 16)] = z16
        return 0

    lax.fori_loop(0, RPT // 16, zbody, 0)
    pltpu.sync_copy(zero_v, acc_sh.at[pl.ds(sid * RPT, RPT)])
    pltpu.sync_copy(src_hbm.at[pl.ds(wid * cw, cw)], idx_v)
    plsc.subcore_barrier()

    def body(j, _):
        pltpu.async_copy(ones_v, acc_sh.at[idx_v.at[j]], dsem, add=True)
        return 0

    lax.fori_loop(0, cw, body, 0)

    def dbody(j, _):
        pltpu.make_async_copy(ones_v, acc_sh.at[idx_v.at[0]], dsem).wait()
        return 0

    lax.fori_loop(0, cw, dbody, 0)
    plsc.subcore_barrier()
    pltpu.sync_copy(acc_sh.at[pl.ds(sid * RPT, RPT)],
                    out_hbm.at[cid, pl.ds(sid * RPT, RPT)])


@functools.partial(
    pl.kernel,
    out_type=jax.ShapeDtypeStruct((NC, NP, D), jnp.float32),
    mesh=_mesh,
    scratch_types=[
        pltpu.VMEM((GPP, CG), jnp.int32),
        pltpu.VMEM((SPP, CS), jnp.int32),
        pltpu.VMEM((NSLOT * CG, D), jnp.float32),
        pltpu.VMEM_SHARED((NP, D), jnp.float32),
        [pltpu.SemaphoreType.DMA] * NSLOT,
        [pltpu.SemaphoreType.DMA] * (NSLOT // 2),
    ],
)
def _hop(g_hbm, src_hbm, dst_hbm, out_hbm,
         src_v, dst_v, ring_v, acc_sh, gsems, ssems):
    cid = lax.axis_index("c")
    sid = lax.axis_index("s")
    wid = cid * NS + sid

    z16 = jnp.zeros((16,), jnp.float32)

    def zrow(r, _):
        for k in range(D // 16):
            ring_v[r, pl.ds(k * 16, 16)] = z16
        return 0

    lax.fori_loop(0, CS, zrow, 0)

    def zcp(cnk, _):
        pltpu.async_copy(ring_v.at[pl.ds(0, CS)],
                         acc_sh.at[pl.ds(sid * RPT + cnk * CS, CS)],
                         gsems[0])
        return 0

    lax.fori_loop(0, RPT // CS, zcp, 0)

    def zwait(cnk, _):
        pltpu.make_async_copy(ring_v.at[pl.ds(0, CS)],
                              acc_sh.at[pl.ds(sid * RPT, CS)],
                              gsems[0]).wait()
        return 0

    lax.fori_loop(0, RPT // CS, zwait, 0)
    plsc.subcore_barrier()

    def slot(s):
        return ring_v.at[pl.ds(s * CG, CG)]

    def gather(c, s):
        pltpu.async_copy(g_hbm.at[src_v.at[c]], slot(s), gsems[s])

    for ph in range(NPH):
        pltpu.sync_copy(src_hbm.at[pl.ds((wid * NPH + ph) * GPP, GPP)],
                        src_v)
        pltpu.sync_copy(dst_hbm.at[pl.ds((wid * NPH + ph) * SPP, SPP)],
                        dst_v)
        for s in range(NSLOT):
            gather(s, s)

        def ring(rr, _):
            for pr in range(NSLOT // 2):
                s0, s1 = 2 * pr, 2 * pr + 1
                c0 = NSLOT * rr + 2 * pr
                k = c0 // 2
                pltpu.make_async_copy(
                    g_hbm.at[src_v.at[c0]], slot(s0), gsems[s0]).wait()
                pltpu.make_async_copy(
                    g_hbm.at[src_v.at[c0]], slot(s1), gsems[s1]).wait()
                pair = ring_v.at[pl.ds(s0 * CG, CS)]
                pltpu.async_copy(pair, acc_sh.at[dst_v.at[k]],
                                 ssems[pr], add=True)
                pltpu.make_async_copy(pair, acc_sh.at[dst_v.at[k]],
                                      ssems[pr]).wait()
                gather(jnp.minimum(c0 + NSLOT, GPP - 2), s0)
                gather(jnp.minimum(c0 + 1 + NSLOT, GPP - 1), s1)
            return 0

        lax.fori_loop(0, GPP // NSLOT, ring, 0)
        for s in range(NSLOT):
            pltpu.make_async_copy(
                g_hbm.at[src_v.at[0]], slot(s), gsems[s]).wait()

    plsc.subcore_barrier()

    def drain(cnk, _):
        base = sid * RPT + cnk * CS
        pltpu.async_copy(acc_sh.at[pl.ds(base, CS)],
                         out_hbm.at[cid, pl.ds(base, CS)], gsems[0])
        return 0

    lax.fori_loop(0, RPT // CS, drain, 0)

    def dwait(cnk, _):
        pltpu.make_async_copy(acc_sh.at[pl.ds(sid * RPT, CS)],
                              out_hbm.at[cid, pl.ds(sid * RPT, CS)],
                              gsems[0]).wait()
        return 0

    lax.fori_loop(0, RPT // CS, dwait, 0)


_RB = 512


def _mm_body(x_ref, w_ref, b_ref, degp_ref, h_ref, g_ref):
    i = pl.program_id(0)
    h = jnp.dot(x_ref[...], w_ref[...],
                preferred_element_type=jnp.float32) + b_ref[...]
    db = (degp_ref[0, pl.ds(i * _RB, _RB)]
          + degp_ref[1, pl.ds(i * _RB, _RB)] + 1.0)
    norm = lax.rsqrt(db).reshape(_RB, 1)
    h_ref[...] = h
    g_ref[...] = h * norm


def _comb_body(accp_ref, h_ref, degp_ref, hn_ref, gn_ref):
    i = pl.program_id(0)
    acc = accp_ref[0] + accp_ref[1]
    db = (degp_ref[0, pl.ds(i * _RB, _RB)]
          + degp_ref[1, pl.ds(i * _RB, _RB)] + 1.0).reshape(_RB, 1)
    norm = lax.rsqrt(db)
    h = h_ref[...]
    hn = acc * norm + h / db
    hn_ref[...] = hn
    gn_ref[...] = hn * norm


def _comb_last_body(accp_ref, h_ref, degp_ref, hn_ref):
    i = pl.program_id(0)
    acc = accp_ref[0] + accp_ref[1]
    db = (degp_ref[0, pl.ds(i * _RB, _RB)]
          + degp_ref[1, pl.ds(i * _RB, _RB)] + 1.0).reshape(_RB, 1)
    hn_ref[...] = acc * lax.rsqrt(db) + h_ref[...] / db


_row_spec = pl.BlockSpec((_RB, D), lambda i: (i, 0))
_degp_spec = pl.BlockSpec((NC, NP), lambda i: (0, 0))
_out2 = [jax.ShapeDtypeStruct((NP, D), jnp.float32)] * 2

_mm_call = pl.pallas_call(
    _mm_body,
    grid=(NP // _RB,),
    in_specs=[
        _row_spec,
        pl.BlockSpec((D, D), lambda i: (0, 0)),
        pl.BlockSpec((1, D), lambda i: (0, 0)),
        _degp_spec,
    ],
    out_specs=[_row_spec, _row_spec],
    out_shape=_out2,
)

_comb_call = pl.pallas_call(
    _comb_body,
    grid=(NP // _RB,),
    in_specs=[
        pl.BlockSpec((NC, _RB, D), lambda i: (0, i, 0)),
        _row_spec,
        _degp_spec,
    ],
    out_specs=[_row_spec, _row_spec],
    out_shape=_out2,
)

_comb_last_call = pl.pallas_call(
    _comb_last_body,
    grid=(NP // _RB,),
    in_specs=[
        pl.BlockSpec((NC, _RB, D), lambda i: (0, i, 0)),
        _row_spec,
        _degp_spec,
    ],
    out_specs=[_row_spec],
    out_shape=[jax.ShapeDtypeStruct((NP, D), jnp.float32)],
)


def kernel(x, edge_index, W, b):
    src = edge_index[0]
    dst = edge_index[1]
    pad = N + jnp.arange(E_PAD - E, dtype=jnp.int32) % (NP - N)
    src_pad = jnp.concatenate([src, pad])
    srcg = src_pad.reshape(E_PAD // CG, CG)
    srcs = src_pad.reshape(E_PAD // CS, CS)
    dstr = jnp.concatenate([dst, pad]).reshape(E_PAD // CS, CS)
    x_pad = jnp.pad(x, ((0, NP - N), (0, 0)))

    degp = _deg(srcs)
    h, g = _mm_call(x_pad, W, b.reshape(1, D), degp)
    accp = _hop(g, srcg, dstr)
    h, g = _comb_call(accp, h, degp)
    accp = _hop(g, srcg, dstr)
    (h,) = _comb_last_call(accp, h, degp)
    return h[:N]


# speedup vs baseline: 1.2681x; 1.0486x over previous
import functools

import jax
import jax.numpy as jnp
from jax import lax
from jax.experimental import pallas as pl
from jax.experimental.pallas import tpu as pltpu
from jax.experimental.pallas import tpu_sc as plsc

N = 10000
E = 320000
D = 128
NC, NS = 2, 16
NW = NC * NS
CG = 64
CS = 128
EPW = 10240
E_PAD = NW * EPW
NPH = 2
GPP = EPW // NPH // CG
SPP = EPW // NPH // CS
NSLOT = 4
NP = 10240
RPT = NP // NS

_mesh = plsc.VectorSubcoreMesh(core_axis_name="c", subcore_axis_name="s")


@functools.partial(
    pl.kernel,
    out_type=jax.ShapeDtypeStruct((NC, NP), jnp.float32),
    mesh=_mesh,
    scratch_types=[
        pltpu.VMEM((EPW // CS, CS), jnp.int32),
        pltpu.VMEM((CS,), jnp.float32),
        pltpu.VMEM((RPT,), jnp.float32),
        pltpu.VMEM_SHARED((NP,), jnp.float32),
        pltpu.SemaphoreType.DMA,
    ],
)
def _deg(src_hbm, out_hbm, idx_v, ones_v, zero_v, acc_sh, dsem):
    cid = lax.axis_index("c")
    sid = lax.axis_index("s")
    wid = cid * NS + sid
    cw = EPW // CS

    one16 = jnp.ones((16,), jnp.float32)
    z16 = jnp.zeros((16,), jnp.float32)
    for k in range(CS // 16):
        ones_v[pl.ds(k * 16, 16)] = one16

    def zbody(k, _):
        zero_v[pl.ds(k * 16, 16)] = z16
        return 0

    lax.fori_loop(0, RPT // 16, zbody, 0)
    pltpu.sync_copy(zero_v, acc_sh.at[pl.ds(sid * RPT, RPT)])
    pltpu.sync_copy(src_hbm.at[pl.ds(wid * cw, cw)], idx_v)
    plsc.subcore_barrier()

    def body(j, _):
        pltpu.async_copy(ones_v, acc_sh.at[idx_v.at[j]], dsem, add=True)
        return 0

    lax.fori_loop(0, cw, body, 0)

    def dbody(j, _):
        pltpu.make_async_copy(ones_v, acc_sh.at[idx_v.at[0]], dsem).wait()
        return 0

    lax.fori_loop(0, cw, dbody, 0)
    plsc.subcore_barrier()
    pltpu.sync_copy(acc_sh.at[pl.ds(sid * RPT, RPT)],
                    out_hbm.at[cid, pl.ds(sid * RPT, RPT)])


@functools.partial(
    pl.kernel,
    out_type=jax.ShapeDtypeStruct((NC, NP, D), jnp.float32),
    mesh=_mesh,
    scratch_types=[
        pltpu.VMEM((GPP, CG), jnp.int32),
        pltpu.VMEM((SPP, CS), jnp.int32),
        pltpu.VMEM((NSLOT * CG, D), jnp.float32),
        pltpu.VMEM_SHARED((NP, D), jnp.float32),
        [pltpu.SemaphoreType.DMA] * NSLOT,
        [pltpu.SemaphoreType.DMA] * (NSLOT // 2),
    ],
)
def _hop(g_hbm, src_hbm, dst_hbm, out_hbm,
         src_v, dst_v, ring_v, acc_sh, gsems, ssems):
    cid = lax.axis_index("c")
    sid = lax.axis_index("s")
    wid = cid * NS + sid

    z16 = jnp.zeros((16,), jnp.float32)

    def zrow(r, _):
        for k in range(D // 16):
            ring_v[r, pl.ds(k * 16, 16)] = z16
        return 0

    lax.fori_loop(0, CS, zrow, 0)

    def zcp(cnk, _):
        pltpu.async_copy(ring_v.at[pl.ds(0, CS)],
                         acc_sh.at[pl.ds(sid * RPT + cnk * CS, CS)],
                         gsems[0])
        return 0

    lax.fori_loop(0, RPT // CS, zcp, 0)

    def zwait(cnk, _):
        pltpu.make_async_copy(ring_v.at[pl.ds(0, CS)],
                              acc_sh.at[pl.ds(sid * RPT, CS)],
                              gsems[0]).wait()
        return 0

    lax.fori_loop(0, RPT // CS, zwait, 0)
    plsc.subcore_barrier()

    def slot(s):
        return ring_v.at[pl.ds(s * CG, CG)]

    def gather(c, s):
        pltpu.async_copy(g_hbm.at[src_v.at[c]], slot(s), gsems[s])

    for ph in range(NPH):
        pltpu.sync_copy(src_hbm.at[pl.ds((wid * NPH + ph) * GPP, GPP)],
                        src_v)
        pltpu.sync_copy(dst_hbm.at[pl.ds((wid * NPH + ph) * SPP, SPP)],
                        dst_v)
        for s in range(NSLOT):
            gather(s, s)

        def ring(rr, _):
            for pr in range(NSLOT // 2):
                s0, s1 = 2 * pr, 2 * pr + 1
                c0 = NSLOT * rr + 2 * pr
                k = c0 // 2
                pltpu.make_async_copy(
                    g_hbm.at[src_v.at[c0]], slot(s0), gsems[s0]).wait()
                pltpu.make_async_copy(
                    g_hbm.at[src_v.at[c0]], slot(s1), gsems[s1]).wait()
                pair = ring_v.at[pl.ds(s0 * CG, CS)]
                pltpu.async_copy(pair, acc_sh.at[dst_v.at[k]],
                                 ssems[pr], add=True)
                pltpu.make_async_copy(pair, acc_sh.at[dst_v.at[k]],
                                      ssems[pr]).wait()
                gather(jnp.minimum(c0 + NSLOT, GPP - 2), s0)
                gather(jnp.minimum(c0 + 1 + NSLOT, GPP - 1), s1)
            return 0

        lax.fori_loop(0, GPP // NSLOT, ring, 0)
        for s in range(NSLOT):
            pltpu.make_async_copy(
                g_hbm.at[src_v.at[0]], slot(s), gsems[s]).wait()

    plsc.subcore_barrier()

    def drain(cnk, _):
        base = sid * RPT + cnk * CS
        pltpu.async_copy(acc_sh.at[pl.ds(base, CS)],
                         out_hbm.at[cid, pl.ds(base, CS)], gsems[0])
        return 0

    lax.fori_loop(0, RPT // CS, drain, 0)

    def dwait(cnk, _):
        pltpu.make_async_copy(acc_sh.at[pl.ds(sid * RPT, CS)],
                              out_hbm.at[cid, pl.ds(sid * RPT, CS)],
                              gsems[0]).wait()
        return 0

    lax.fori_loop(0, RPT // CS, dwait, 0)


_RB = 1024


def _mm_body(x_ref, w_ref, b_ref, degp_ref, h_ref, g_ref):
    i = pl.program_id(0)
    h = jnp.dot(x_ref[...], w_ref[...],
                preferred_element_type=jnp.float32) + b_ref[...]
    db = (degp_ref[0, pl.ds(i * _RB, _RB)]
          + degp_ref[1, pl.ds(i * _RB, _RB)] + 1.0)
    norm = lax.rsqrt(db).reshape(_RB, 1)
    h_ref[...] = h
    g_ref[...] = h * norm


def _comb_body(accp_ref, h_ref, degp_ref, hn_ref, gn_ref):
    i = pl.program_id(0)
    acc = accp_ref[0] + accp_ref[1]
    db = (degp_ref[0, pl.ds(i * _RB, _RB)]
          + degp_ref[1, pl.ds(i * _RB, _RB)] + 1.0).reshape(_RB, 1)
    norm = lax.rsqrt(db)
    h = h_ref[...]
    hn = acc * norm + h / db
    hn_ref[...] = hn
    gn_ref[...] = hn * norm


def _comb_last_body(accp_ref, h_ref, degp_ref, hn_ref):
    i = pl.program_id(0)
    acc = accp_ref[0] + accp_ref[1]
    db = (degp_ref[0, pl.ds(i * _RB, _RB)]
          + degp_ref[1, pl.ds(i * _RB, _RB)] + 1.0).reshape(_RB, 1)
    hn_ref[...] = acc * lax.rsqrt(db) + h_ref[...] / db


_row_spec = pl.BlockSpec((_RB, D), lambda i: (i, 0))
_degp_spec = pl.BlockSpec((NC, NP), lambda i: (0, 0))
_out2 = [jax.ShapeDtypeStruct((NP, D), jnp.float32)] * 2

_mm_call = pl.pallas_call(
    _mm_body,
    grid=(NP // _RB,),
    in_specs=[
        _row_spec,
        pl.BlockSpec((D, D), lambda i: (0, 0)),
        pl.BlockSpec((1, D), lambda i: (0, 0)),
        _degp_spec,
    ],
    out_specs=[_row_spec, _row_spec],
    out_shape=_out2,
)

_comb_call = pl.pallas_call(
    _comb_body,
    grid=(NP // _RB,),
    in_specs=[
        pl.BlockSpec((NC, _RB, D), lambda i: (0, i, 0)),
        _row_spec,
        _degp_spec,
    ],
    out_specs=[_row_spec, _row_spec],
    out_shape=_out2,
)

_comb_last_call = pl.pallas_call(
    _comb_last_body,
    grid=(NP // _RB,),
    in_specs=[
        pl.BlockSpec((NC, _RB, D), lambda i: (0, i, 0)),
        _row_spec,
        _degp_spec,
    ],
    out_specs=[_row_spec],
    out_shape=[jax.ShapeDtypeStruct((NP, D), jnp.float32)],
)


def kernel(x, edge_index, W, b):
    src = edge_index[0]
    dst = edge_index[1]
    pad = N + jnp.arange(E_PAD - E, dtype=jnp.int32) % (NP - N)
    src_pad = jnp.concatenate([src, pad])
    srcg = src_pad.reshape(E_PAD // CG, CG)
    srcs = src_pad.reshape(E_PAD // CS, CS)
    dstr = jnp.concatenate([dst, pad]).reshape(E_PAD // CS, CS)
    x_pad = jnp.pad(x, ((0, NP - N), (0, 0)))

    degp = _deg(srcs)
    h, g = _mm_call(x_pad, W, b.reshape(1, D), degp)
    accp = _hop(g, srcg, dstr)
    h, g = _comb_call(accp, h, degp)
    accp = _hop(g, srcg, dstr)
    (h,) = _comb_last_call(accp, h, degp)
    return h[:N]


# speedup vs baseline: 1.2946x; 1.0209x over previous
import functools

import jax
import jax.numpy as jnp
from jax import lax
from jax.experimental import pallas as pl
from jax.experimental.pallas import tpu as pltpu
from jax.experimental.pallas import tpu_sc as plsc

N = 10000
E = 320000
D = 128
NC, NS = 2, 16
NW = NC * NS
CG = 64
CS = 128
EPW = 10240
E_PAD = NW * EPW
NPH = 2
GPP = EPW // NPH // CG
SPP = EPW // NPH // CS
NSLOT = 4
NP = 10240
RPT = NP // NS

_mesh = plsc.VectorSubcoreMesh(core_axis_name="c", subcore_axis_name="s")


@functools.partial(
    pl.kernel,
    out_type=jax.ShapeDtypeStruct((NC, NP), jnp.float32),
    mesh=_mesh,
    scratch_types=[
        pltpu.VMEM((EPW // CS, CS), jnp.int32),
        pltpu.VMEM((CS,), jnp.float32),
        pltpu.VMEM((RPT,), jnp.float32),
        pltpu.VMEM_SHARED((NP,), jnp.float32),
        pltpu.SemaphoreType.DMA,
    ],
)
def _deg(src_hbm, out_hbm, idx_v, ones_v, zero_v, acc_sh, dsem):
    cid = lax.axis_index("c")
    sid = lax.axis_index("s")
    wid = cid * NS + sid
    cw = EPW // CS

    one16 = jnp.ones((16,), jnp.float32)
    z16 = jnp.zeros((16,), jnp.float32)
    for k in range(CS // 16):
        ones_v[pl.ds(k * 16, 16)] = one16

    def zbody(k, _):
        zero_v[pl.ds(k * 16, 16)] = z16
        return 0

    lax.fori_loop(0, RPT // 16, zbody, 0)
    pltpu.sync_copy(zero_v, acc_sh.at[pl.ds(sid * RPT, RPT)])
    pltpu.sync_copy(src_hbm.at[pl.ds(wid * cw, cw)], idx_v)
    plsc.subcore_barrier()

    def body(j, _):
        pltpu.async_copy(ones_v, acc_sh.at[idx_v.at[j]], dsem, add=True)
        return 0

    lax.fori_loop(0, cw, body, 0)

    def dbody(j, _):
        pltpu.make_async_copy(ones_v, acc_sh.at[idx_v.at[0]], dsem).wait()
        return 0

    lax.fori_loop(0, cw, dbody, 0)
    plsc.subcore_barrier()
    pltpu.sync_copy(acc_sh.at[pl.ds(sid * RPT, RPT)],
                    out_hbm.at[cid, pl.ds(sid * RPT, RPT)])


@functools.partial(
    pl.kernel,
    out_type=jax.ShapeDtypeStruct((NC, NP, D), jnp.float32),
    mesh=_mesh,
    scratch_types=[
        pltpu.VMEM((GPP, CG), jnp.int32),
        pltpu.VMEM((SPP, CS), jnp.int32),
        pltpu.VMEM((NSLOT * CG, D), jnp.float32),
        pltpu.VMEM_SHARED((NP, D), jnp.float32),
        [pltpu.SemaphoreType.DMA] * NSLOT,
        [pltpu.SemaphoreType.DMA] * (NSLOT // 2),
    ],
)
def _hop(g_hbm, src_hbm, dst_hbm, out_hbm,
         src_v, dst_v, ring_v, acc_sh, gsems, ssems):
    cid = lax.axis_index("c")
    sid = lax.axis_index("s")
    wid = cid * NS + sid

    z16 = jnp.zeros((16,), jnp.float32)

    def zrow(r, _):
        for k in range(D // 16):
            ring_v[r, pl.ds(k * 16, 16)] = z16
        return 0

    lax.fori_loop(0, CS, zrow, 0)

    def zcp(cnk, _):
        pltpu.async_copy(ring_v.at[pl.ds(0, CS)],
                         acc_sh.at[pl.ds(sid * RPT + cnk * CS, CS)],
                         gsems[0])
        return 0

    lax.fori_loop(0, RPT // CS, zcp, 0)

    def zwait(cnk, _):
        pltpu.make_async_copy(ring_v.at[pl.ds(0, CS)],
                              acc_sh.at[pl.ds(sid * RPT, CS)],
                              gsems[0]).wait()
        return 0

    lax.fori_loop(0, RPT // CS, zwait, 0)
    plsc.subcore_barrier()

    def slot(s):
        return ring_v.at[pl.ds(s * CG, CG)]

    def gather(c, s):
        pltpu.async_copy(g_hbm.at[src_v.at[c]], slot(s), gsems[s])

    for ph in range(NPH):
        pltpu.sync_copy(src_hbm.at[pl.ds((wid * NPH + ph) * GPP, GPP)],
                        src_v)
        pltpu.sync_copy(dst_hbm.at[pl.ds((wid * NPH + ph) * SPP, SPP)],
                        dst_v)
        for s in range(NSLOT):
            gather(s, s)

        def ring(rr, _):
            for pr in range(NSLOT // 2):
                s0, s1 = 2 * pr, 2 * pr + 1
                c0 = NSLOT * rr + 2 * pr
                k = c0 // 2
                pltpu.make_async_copy(
                    g_hbm.at[src_v.at[c0]], slot(s0), gsems[s0]).wait()
                pltpu.make_async_copy(
                    g_hbm.at[src_v.at[c0]], slot(s1), gsems[s1]).wait()
                pair = ring_v.at[pl.ds(s0 * CG, CS)]
                pltpu.async_copy(pair, acc_sh.at[dst_v.at[k]],
                                 ssems[pr], add=True)
                pltpu.make_async_copy(pair, acc_sh.at[dst_v.at[k]],
                                      ssems[pr]).wait()
                gather(jnp.minimum(c0 + NSLOT, GPP - 2), s0)
                gather(jnp.minimum(c0 + 1 + NSLOT, GPP - 1), s1)
            return 0

        lax.fori_loop(0, GPP // NSLOT, ring, 0)
        for s in range(NSLOT):
            pltpu.make_async_copy(
                g_hbm.at[src_v.at[0]], slot(s), gsems[s]).wait()

    plsc.subcore_barrier()

    def drain(cnk, _):
        base = sid * RPT + cnk * CS
        pltpu.async_copy(acc_sh.at[pl.ds(base, CS)],
                         out_hbm.at[cid, pl.ds(base, CS)], gsems[0])
        return 0

    lax.fori_loop(0, RPT // CS, drain, 0)

    def dwait(cnk, _):
        pltpu.make_async_copy(acc_sh.at[pl.ds(sid * RPT, CS)],
                              out_hbm.at[cid, pl.ds(sid * RPT, CS)],
                              gsems[0]).wait()
        return 0

    lax.fori_loop(0, RPT // CS, dwait, 0)


_RB = 2048


def _mm_body(x_ref, w_ref, b_ref, degp_ref, h_ref, g_ref):
    i = pl.program_id(0)
    h = jnp.dot(x_ref[...], w_ref[...],
                preferred_element_type=jnp.float32) + b_ref[...]
    db = (degp_ref[0, pl.ds(i * _RB, _RB)]
          + degp_ref[1, pl.ds(i * _RB, _RB)] + 1.0)
    norm = lax.rsqrt(db).reshape(_RB, 1)
    h_ref[...] = h
    g_ref[...] = h * norm


def _comb_body(accp_ref, h_ref, degp_ref, hn_ref, gn_ref):
    i = pl.program_id(0)
    acc = accp_ref[0] + accp_ref[1]
    db = (degp_ref[0, pl.ds(i * _RB, _RB)]
          + degp_ref[1, pl.ds(i * _RB, _RB)] + 1.0).reshape(_RB, 1)
    norm = lax.rsqrt(db)
    h = h_ref[...]
    hn = acc * norm + h / db
    hn_ref[...] = hn
    gn_ref[...] = hn * norm


def _comb_last_body(accp_ref, h_ref, degp_ref, hn_ref):
    i = pl.program_id(0)
    acc = accp_ref[0] + accp_ref[1]
    db = (degp_ref[0, pl.ds(i * _RB, _RB)]
          + degp_ref[1, pl.ds(i * _RB, _RB)] + 1.0).reshape(_RB, 1)
    hn_ref[...] = acc * lax.rsqrt(db) + h_ref[...] / db


_row_spec = pl.BlockSpec((_RB, D), lambda i: (i, 0))
_degp_spec = pl.BlockSpec((NC, NP), lambda i: (0, 0))
_out2 = [jax.ShapeDtypeStruct((NP, D), jnp.float32)] * 2

_mm_call = pl.pallas_call(
    _mm_body,
    grid=(NP // _RB,),
    in_specs=[
        _row_spec,
        pl.BlockSpec((D, D), lambda i: (0, 0)),
        pl.BlockSpec((1, D), lambda i: (0, 0)),
        _degp_spec,
    ],
    out_specs=[_row_spec, _row_spec],
    out_shape=_out2,
)

_comb_call = pl.pallas_call(
    _comb_body,
    grid=(NP // _RB,),
    in_specs=[
        pl.BlockSpec((NC, _RB, D), lambda i: (0, i, 0)),
        _row_spec,
        _degp_spec,
    ],
    out_specs=[_row_spec, _row_spec],
    out_shape=_out2,
)

_comb_last_call = pl.pallas_call(
    _comb_last_body,
    grid=(NP // _RB,),
    in_specs=[
        pl.BlockSpec((NC, _RB, D), lambda i: (0, i, 0)),
        _row_spec,
        _degp_spec,
    ],
    out_specs=[_row_spec],
    out_shape=[jax.ShapeDtypeStruct((NP, D), jnp.float32)],
)


def kernel(x, edge_index, W, b):
    src = edge_index[0]
    dst = edge_index[1]
    pad = N + jnp.arange(E_PAD - E, dtype=jnp.int32) % (NP - N)
    src_pad = jnp.concatenate([src, pad])
    srcg = src_pad.reshape(E_PAD // CG, CG)
    srcs = src_pad.reshape(E_PAD // CS, CS)
    dstr = jnp.concatenate([dst, pad]).reshape(E_PAD // CS, CS)
    x_pad = jnp.pad(x, ((0, NP - N), (0, 0)))

    degp = _deg(srcs)
    h, g = _mm_call(x_pad, W, b.reshape(1, D), degp)
    accp = _hop(g, srcg, dstr)
    h, g = _comb_call(accp, h, degp)
    accp = _hop(g, srcg, dstr)
    (h,) = _comb_last_call(accp, h, degp)
    return h[:N]


# speedup vs baseline: 1.3157x; 1.0163x over previous
import functools

import jax
import jax.numpy as jnp
from jax import lax
from jax.experimental import pallas as pl
from jax.experimental.pallas import tpu as pltpu
from jax.experimental.pallas import tpu_sc as plsc

N = 10000
E = 320000
D = 128
NC, NS = 2, 16
NW = NC * NS
CG = 64
CS = 128
EPW = 10240
E_PAD = NW * EPW
NPH = 2
GPP = EPW // NPH // CG
SPP = EPW // NPH // CS
NSLOT = 4
NP = 10240
RPT = NP // NS

_mesh = plsc.VectorSubcoreMesh(core_axis_name="c", subcore_axis_name="s")


@functools.partial(
    pl.kernel,
    out_type=jax.ShapeDtypeStruct((NC, NP), jnp.float32),
    mesh=_mesh,
    scratch_types=[
        pltpu.VMEM((EPW // CS, CS), jnp.int32),
        pltpu.VMEM((CS,), jnp.float32),
        pltpu.VMEM((RPT,), jnp.float32),
        pltpu.VMEM_SHARED((NP,), jnp.float32),
        pltpu.SemaphoreType.DMA,
    ],
)
def _deg(src_hbm, out_hbm, idx_v, ones_v, zero_v, acc_sh, dsem):
    cid = lax.axis_index("c")
    sid = lax.axis_index("s")
    wid = cid * NS + sid
    cw = EPW // CS

    one16 = jnp.ones((16,), jnp.float32)
    z16 = jnp.zeros((16,), jnp.float32)
    for k in range(CS // 16):
        ones_v[pl.ds(k * 16, 16)] = one16

    def zbody(k, _):
        zero_v[pl.ds(k * 16, 16)] = z16
        return 0

    lax.fori_loop(0, RPT // 16, zbody, 0)
    pltpu.sync_copy(zero_v, acc_sh.at[pl.ds(sid * RPT, RPT)])
    pltpu.sync_copy(src_hbm.at[pl.ds(wid * cw, cw)], idx_v)
    plsc.subcore_barrier()

    def body(j, _):
        pltpu.async_copy(ones_v, acc_sh.at[idx_v.at[j]], dsem, add=True)
        return 0

    lax.fori_loop(0, cw, body, 0)

    def dbody(j, _):
        pltpu.make_async_copy(ones_v, acc_sh.at[idx_v.at[0]], dsem).wait()
        return 0

    lax.fori_loop(0, cw, dbody, 0)
    plsc.subcore_barrier()
    pltpu.sync_copy(acc_sh.at[pl.ds(sid * RPT, RPT)],
                    out_hbm.at[cid, pl.ds(sid * RPT, RPT)])


@functools.partial(
    pl.kernel,
    out_type=jax.ShapeDtypeStruct((NC, NP, D), jnp.float32),
    mesh=_mesh,
    scratch_types=[
        pltpu.VMEM((GPP, CG), jnp.int32),
        pltpu.VMEM((SPP, CS), jnp.int32),
        pltpu.VMEM((NSLOT * CG, D), jnp.float32),
        pltpu.VMEM_SHARED((NP, D), jnp.float32),
        [pltpu.SemaphoreType.DMA] * NSLOT,
        [pltpu.SemaphoreType.DMA] * (NSLOT // 2),
    ],
)
def _hop(g_hbm, src_hbm, dst_hbm, out_hbm,
         src_v, dst_v, ring_v, acc_sh, gsems, ssems):
    cid = lax.axis_index("c")
    sid = lax.axis_index("s")
    wid = cid * NS + sid

    z16 = jnp.zeros((16,), jnp.float32)

    def zrow(r, _):
        for k in range(D // 16):
            ring_v[r, pl.ds(k * 16, 16)] = z16
        return 0

    lax.fori_loop(0, CS, zrow, 0)

    def zcp(cnk, _):
        pltpu.async_copy(ring_v.at[pl.ds(0, CS)],
                         acc_sh.at[pl.ds(sid * RPT + cnk * CS, CS)],
                         gsems[0])
        return 0

    lax.fori_loop(0, RPT // CS, zcp, 0)

    def zwait(cnk, _):
        pltpu.make_async_copy(ring_v.at[pl.ds(0, CS)],
                              acc_sh.at[pl.ds(sid * RPT, CS)],
                              gsems[0]).wait()
        return 0

    lax.fori_loop(0, RPT // CS, zwait, 0)
    plsc.subcore_barrier()

    def slot(s):
        return ring_v.at[pl.ds(s * CG, CG)]

    def gather(c, s):
        pltpu.async_copy(g_hbm.at[src_v.at[c]], slot(s), gsems[s])

    for ph in range(NPH):
        pltpu.sync_copy(src_hbm.at[pl.ds((wid * NPH + ph) * GPP, GPP)],
                        src_v)
        pltpu.sync_copy(dst_hbm.at[pl.ds((wid * NPH + ph) * SPP, SPP)],
                        dst_v)
        for s in range(NSLOT):
            gather(s, s)

        def ring(rr, _):
            for pr in range(NSLOT // 2):
                s0, s1 = 2 * pr, 2 * pr + 1
                c0 = NSLOT * rr + 2 * pr
                k = c0 // 2
                pltpu.make_async_copy(
                    g_hbm.at[src_v.at[c0]], slot(s0), gsems[s0]).wait()
                pltpu.make_async_copy(
                    g_hbm.at[src_v.at[c0]], slot(s1), gsems[s1]).wait()
                pair = ring_v.at[pl.ds(s0 * CG, CS)]
                pltpu.async_copy(pair, acc_sh.at[dst_v.at[k]],
                                 ssems[pr], add=True)
                pltpu.make_async_copy(pair, acc_sh.at[dst_v.at[k]],
                                      ssems[pr]).wait()
                gather(jnp.minimum(c0 + NSLOT, GPP - 2), s0)
                gather(jnp.minimum(c0 + 1 + NSLOT, GPP - 1), s1)
            return 0

        lax.fori_loop(0, GPP // NSLOT, ring, 0)
        for s in range(NSLOT):
            pltpu.make_async_copy(
                g_hbm.at[src_v.at[0]], slot(s), gsems[s]).wait()

    plsc.subcore_barrier()

    def drain(cnk, _):
        base = sid * RPT + cnk * CS
        pltpu.async_copy(acc_sh.at[pl.ds(base, CS)],
                         out_hbm.at[cid, pl.ds(base, CS)], gsems[0])
        return 0

    lax.fori_loop(0, RPT // CS, drain, 0)

    def dwait(cnk, _):
        pltpu.make_async_copy(acc_sh.at[pl.ds(sid * RPT, CS)],
                              out_hbm.at[cid, pl.ds(sid * RPT, CS)],
                              gsems[0]).wait()
        return 0

    lax.fori_loop(0, RPT // CS, dwait, 0)


_RB = 5120


def _mm_body(x_ref, w_ref, b_ref, degp_ref, h_ref, g_ref):
    i = pl.program_id(0)
    h = jnp.dot(x_ref[...], w_ref[...],
                preferred_element_type=jnp.float32) + b_ref[...]
    db = (degp_ref[0, pl.ds(i * _RB, _RB)]
          + degp_ref[1, pl.ds(i * _RB, _RB)] + 1.0)
    norm = lax.rsqrt(db).reshape(_RB, 1)
    h_ref[...] = h
    g_ref[...] = h * norm


def _comb_body(accp_ref, h_ref, degp_ref, hn_ref, gn_ref):
    i = pl.program_id(0)
    acc = accp_ref[0] + accp_ref[1]
    db = (degp_ref[0, pl.ds(i * _RB, _RB)]
          + degp_ref[1, pl.ds(i * _RB, _RB)] + 1.0).reshape(_RB, 1)
    norm = lax.rsqrt(db)
    h = h_ref[...]
    hn = acc * norm + h / db
    hn_ref[...] = hn
    gn_ref[...] = hn * norm


def _comb_last_body(accp_ref, h_ref, degp_ref, hn_ref):
    i = pl.program_id(0)
    acc = accp_ref[0] + accp_ref[1]
    db = (degp_ref[0, pl.ds(i * _RB, _RB)]
          + degp_ref[1, pl.ds(i * _RB, _RB)] + 1.0).reshape(_RB, 1)
    hn_ref[...] = acc * lax.rsqrt(db) + h_ref[...] / db


_row_spec = pl.BlockSpec((_RB, D), lambda i: (i, 0))
_degp_spec = pl.BlockSpec((NC, NP), lambda i: (0, 0))
_out2 = [jax.ShapeDtypeStruct((NP, D), jnp.float32)] * 2

_mm_call = pl.pallas_call(
    _mm_body,
    grid=(NP // _RB,),
    in_specs=[
        _row_spec,
        pl.BlockSpec((D, D), lambda i: (0, 0)),
        pl.BlockSpec((1, D), lambda i: (0, 0)),
        _degp_spec,
    ],
    out_specs=[_row_spec, _row_spec],
    out_shape=_out2,
)

_comb_call = pl.pallas_call(
    _comb_body,
    grid=(NP // _RB,),
    in_specs=[
        pl.BlockSpec((NC, _RB, D), lambda i: (0, i, 0)),
        _row_spec,
        _degp_spec,
    ],
    out_specs=[_row_spec, _row_spec],
    out_shape=_out2,
)

_comb_last_call = pl.pallas_call(
    _comb_last_body,
    grid=(NP // _RB,),
    in_specs=[
        pl.BlockSpec((NC, _RB, D), lambda i: (0, i, 0)),
        _row_spec,
        _degp_spec,
    ],
    out_specs=[_row_spec],
    out_shape=[jax.ShapeDtypeStruct((NP, D), jnp.float32)],
)


def kernel(x, edge_index, W, b):
    src = edge_index[0]
    dst = edge_index[1]
    pad = N + jnp.arange(E_PAD - E, dtype=jnp.int32) % (NP - N)
    src_pad = jnp.concatenate([src, pad])
    srcg = src_pad.reshape(E_PAD // CG, CG)
    srcs = src_pad.reshape(E_PAD // CS, CS)
    dstr = jnp.concatenate([dst, pad]).reshape(E_PAD // CS, CS)
    x_pad = jnp.pad(x, ((0, NP - N), (0, 0)))

    degp = _deg(srcs)
    h, g = _mm_call(x_pad, W, b.reshape(1, D), degp)
    accp = _hop(g, srcg, dstr)
    h, g = _comb_call(accp, h, degp)
    accp = _hop(g, srcg, dstr)
    (h,) = _comb_last_call(accp, h, degp)
    return h[:N]


# speedup vs baseline: 1.3313x; 1.0119x over previous
import functools

import jax
import jax.numpy as jnp
from jax import lax
from jax.experimental import pallas as pl
from jax.experimental.pallas import tpu as pltpu
from jax.experimental.pallas import tpu_sc as plsc

N = 10000
E = 320000
D = 128
NC, NS = 2, 16
NW = NC * NS
CG = 64
CS = 128
EPW = 10240
E_PAD = NW * EPW
NPH = 2
GPP = EPW // NPH // CG
SPP = EPW // NPH // CS
NSLOT = 4
NP = 10240
RPT = NP // NS

_mesh = plsc.VectorSubcoreMesh(core_axis_name="c", subcore_axis_name="s")


@functools.partial(
    pl.kernel,
    out_type=jax.ShapeDtypeStruct((NC, NP), jnp.float32),
    mesh=_mesh,
    scratch_types=[
        pltpu.VMEM((EPW // CS, CS), jnp.int32),
        pltpu.VMEM((CS,), jnp.float32),
        pltpu.VMEM((RPT,), jnp.float32),
        pltpu.VMEM_SHARED((NP,), jnp.float32),
        pltpu.SemaphoreType.DMA,
    ],
)
def _deg(src_hbm, out_hbm, idx_v, ones_v, zero_v, acc_sh, dsem):
    cid = lax.axis_index("c")
    sid = lax.axis_index("s")
    wid = cid * NS + sid
    cw = EPW // CS

    one16 = jnp.ones((16,), jnp.float32)
    z16 = jnp.zeros((16,), jnp.float32)
    for k in range(CS // 16):
        ones_v[pl.ds(k * 16, 16)] = one16

    def zbody(k, _):
        zero_v[pl.ds(k * 16, 16)] = z16
        return 0

    lax.fori_loop(0, RPT // 16, zbody, 0)
    pltpu.sync_copy(zero_v, acc_sh.at[pl.ds(sid * RPT, RPT)])
    pltpu.sync_copy(src_hbm.at[pl.ds(wid * cw, cw)], idx_v)
    plsc.subcore_barrier()

    def body(j, _):
        pltpu.async_copy(ones_v, acc_sh.at[idx_v.at[j]], dsem, add=True)
        return 0

    lax.fori_loop(0, cw, body, 0)

    def dbody(j, _):
        pltpu.make_async_copy(ones_v, acc_sh.at[idx_v.at[0]], dsem).wait()
        return 0

    lax.fori_loop(0, cw, dbody, 0)
    plsc.subcore_barrier()
    pltpu.sync_copy(acc_sh.at[pl.ds(sid * RPT, RPT)],
                    out_hbm.at[cid, pl.ds(sid * RPT, RPT)])


@functools.partial(
    pl.kernel,
    out_type=jax.ShapeDtypeStruct((NC, NP, D), jnp.float32),
    mesh=_mesh,
    scratch_types=[
        pltpu.VMEM((GPP, CG), jnp.int32),
        pltpu.VMEM((SPP, CS), jnp.int32),
        pltpu.VMEM((NSLOT * CG, D), jnp.float32),
        pltpu.VMEM_SHARED((NP, D), jnp.float32),
        [pltpu.SemaphoreType.DMA] * NSLOT,
        [pltpu.SemaphoreType.DMA] * (NSLOT // 2),
    ],
)
def _hop(g_hbm, src_hbm, dst_hbm, out_hbm,
         src_v, dst_v, ring_v, acc_sh, gsems, ssems):
    cid = lax.axis_index("c")
    sid = lax.axis_index("s")
    wid = cid * NS + sid

    pltpu.async_copy(src_hbm.at[pl.ds(wid * NPH * GPP, GPP)], src_v,
                     ssems[0])
    pltpu.async_copy(dst_hbm.at[pl.ds(wid * NPH * SPP, SPP)], dst_v,
                     ssems[1])

    z16 = jnp.zeros((16,), jnp.float32)

    def zrow(r, _):
        for k in range(D // 16):
            ring_v[r, pl.ds(k * 16, 16)] = z16
        return 0

    lax.fori_loop(0, CS, zrow, 0)

    def zcp(cnk, _):
        pltpu.async_copy(ring_v.at[pl.ds(0, CS)],
                         acc_sh.at[pl.ds(sid * RPT + cnk * CS, CS)],
                         gsems[0])
        return 0

    lax.fori_loop(0, RPT // CS, zcp, 0)

    def zwait(cnk, _):
        pltpu.make_async_copy(ring_v.at[pl.ds(0, CS)],
                              acc_sh.at[pl.ds(sid * RPT, CS)],
                              gsems[0]).wait()
        return 0

    lax.fori_loop(0, RPT // CS, zwait, 0)
    plsc.subcore_barrier()

    def slot(s):
        return ring_v.at[pl.ds(s * CG, CG)]

    def gather(c, s):
        pltpu.async_copy(g_hbm.at[src_v.at[c]], slot(s), gsems[s])

    for ph in range(NPH):
        if ph == 0:
            pltpu.make_async_copy(
                src_hbm.at[pl.ds(wid * NPH * GPP, GPP)], src_v,
                ssems[0]).wait()
            pltpu.make_async_copy(
                dst_hbm.at[pl.ds(wid * NPH * SPP, SPP)], dst_v,
                ssems[1]).wait()
        else:
            pltpu.sync_copy(src_hbm.at[pl.ds((wid * NPH + ph) * GPP, GPP)],
                            src_v)
            pltpu.sync_copy(dst_hbm.at[pl.ds((wid * NPH + ph) * SPP, SPP)],
                            dst_v)
        for s in range(NSLOT):
            gather(s, s)

        def ring(rr, _):
            for pr in range(NSLOT // 2):
                s0, s1 = 2 * pr, 2 * pr + 1
                c0 = NSLOT * rr + 2 * pr
                k = c0 // 2
                pltpu.make_async_copy(
                    g_hbm.at[src_v.at[c0]], slot(s0), gsems[s0]).wait()
                pltpu.make_async_copy(
                    g_hbm.at[src_v.at[c0]], slot(s1), gsems[s1]).wait()
                pair = ring_v.at[pl.ds(s0 * CG, CS)]
                pltpu.async_copy(pair, acc_sh.at[dst_v.at[k]],
                                 ssems[pr], add=True)
                pltpu.make_async_copy(pair, acc_sh.at[dst_v.at[k]],
                                      ssems[pr]).wait()
                gather(jnp.minimum(c0 + NSLOT, GPP - 2), s0)
                gather(jnp.minimum(c0 + 1 + NSLOT, GPP - 1), s1)
            return 0

        lax.fori_loop(0, GPP // NSLOT, ring, 0)
        for s in range(NSLOT):
            pltpu.make_async_copy(
                g_hbm.at[src_v.at[0]], slot(s), gsems[s]).wait()

    plsc.subcore_barrier()

    def drain(cnk, _):
        base = sid * RPT + cnk * CS
        pltpu.async_copy(acc_sh.at[pl.ds(base, CS)],
                         out_hbm.at[cid, pl.ds(base, CS)], gsems[0])
        return 0

    lax.fori_loop(0, RPT // CS, drain, 0)

    def dwait(cnk, _):
        pltpu.make_async_copy(acc_sh.at[pl.ds(sid * RPT, CS)],
                              out_hbm.at[cid, pl.ds(sid * RPT, CS)],
                              gsems[0]).wait()
        return 0

    lax.fori_loop(0, RPT // CS, dwait, 0)


_RB = 5120


def _mm_body(x_ref, w_ref, b_ref, degp_ref, h_ref, g_ref):
    i = pl.program_id(0)
    h = jnp.dot(x_ref[...], w_ref[...],
                preferred_element_type=jnp.float32) + b_ref[...]
    db = (degp_ref[0, pl.ds(i * _RB, _RB)]
          + degp_ref[1, pl.ds(i * _RB, _RB)] + 1.0)
    norm = lax.rsqrt(db).reshape(_RB, 1)
    h_ref[...] = h
    g_ref[...] = h * norm


def _comb_body(accp_ref, h_ref, degp_ref, hn_ref, gn_ref):
    i = pl.program_id(0)
    acc = accp_ref[0] + accp_ref[1]
    db = (degp_ref[0, pl.ds(i * _RB, _RB)]
          + degp_ref[1, pl.ds(i * _RB, _RB)] + 1.0).reshape(_RB, 1)
    norm = lax.rsqrt(db)
    h = h_ref[...]
    hn = acc * norm + h / db
    hn_ref[...] = hn
    gn_ref[...] = hn * norm


def _comb_last_body(accp_ref, h_ref, degp_ref, hn_ref):
    i = pl.program_id(0)
    acc = accp_ref[0] + accp_ref[1]
    db = (degp_ref[0, pl.ds(i * _RB, _RB)]
          + degp_ref[1, pl.ds(i * _RB, _RB)] + 1.0).reshape(_RB, 1)
    hn_ref[...] = acc * lax.rsqrt(db) + h_ref[...] / db


_row_spec = pl.BlockSpec((_RB, D), lambda i: (i, 0))
_degp_spec = pl.BlockSpec((NC, NP), lambda i: (0, 0))
_out2 = [jax.ShapeDtypeStruct((NP, D), jnp.float32)] * 2

_mm_call = pl.pallas_call(
    _mm_body,
    grid=(NP // _RB,),
    in_specs=[
        _row_spec,
        pl.BlockSpec((D, D), lambda i: (0, 0)),
        pl.BlockSpec((1, D), lambda i: (0, 0)),
        _degp_spec,
    ],
    out_specs=[_row_spec, _row_spec],
    out_shape=_out2,
)

_comb_call = pl.pallas_call(
    _comb_body,
    grid=(NP // _RB,),
    in_specs=[
        pl.BlockSpec((NC, _RB, D), lambda i: (0, i, 0)),
        _row_spec,
        _degp_spec,
    ],
    out_specs=[_row_spec, _row_spec],
    out_shape=_out2,
)

_comb_last_call = pl.pallas_call(
    _comb_last_body,
    grid=(NP // _RB,),
    in_specs=[
        pl.BlockSpec((NC, _RB, D), lambda i: (0, i, 0)),
        _row_spec,
        _degp_spec,
    ],
    out_specs=[_row_spec],
    out_shape=[jax.ShapeDtypeStruct((NP, D), jnp.float32)],
)


def kernel(x, edge_index, W, b):
    src = edge_index[0]
    dst = edge_index[1]
    pad = N + jnp.arange(E_PAD - E, dtype=jnp.int32) % (NP - N)
    src_pad = jnp.concatenate([src, pad])
    srcg = src_pad.reshape(E_PAD // CG, CG)
    srcs = src_pad.reshape(E_PAD // CS, CS)
    dstr = jnp.concatenate([dst, pad]).reshape(E_PAD // CS, CS)
    x_pad = jnp.pad(x, ((0, NP - N), (0, 0)))

    degp = _deg(srcs)
    h, g = _mm_call(x_pad, W, b.reshape(1, D), degp)
    accp = _hop(g, srcg, dstr)
    h, g = _comb_call(accp, h, degp)
    accp = _hop(g, srcg, dstr)
    (h,) = _comb_last_call(accp, h, degp)
    return h[:N]
